# Initial kernel scaffold; baseline (speedup 1.0000x reference)
import jax
import jax.numpy as jnp
from jax.experimental import pallas as pl


def kernel(x, laplacian, weight, bias):
    raise NotImplementedError("write your pallas kernel here")


# speedup vs baseline: 1.2695x; 1.2695x over previous
import functools

import jax
import jax.numpy as jnp
from jax.experimental import pallas as pl
from jax.experimental.pallas import tpu as pltpu


def _pick_tile(v: int) -> int:
    for t in (400, 500, 256, 250, 200, 128, 100, 64, 50, 40, 32, 16, 8):
        if v % t == 0:
            return t
    return v


def _stage1(l_ref, x0_ref, x1_ref):
    lb = l_ref[...].astype(jnp.bfloat16)
    acc = jnp.dot(lb, x0_ref[...], preferred_element_type=jnp.float32)
    x1_ref[...] = acc.astype(jnp.bfloat16)


def _stage2(l_ref, x0_ref, x1_ref, w_ref, b_ref, out_ref, *, tile, n_b, cin):
    i = pl.program_id(0)
    lb = l_ref[...].astype(jnp.bfloat16)
    x2 = 2.0 * jnp.dot(lb, x1_ref[...], preferred_element_type=jnp.float32)
    x0t = x0_ref[...]
    x2 = x2 - x0t.astype(jnp.float32)
    x2b = x2.astype(jnp.bfloat16)
    x1t = x1_ref[pl.ds(i * tile, tile), :]
    outs = []
    for b in range(n_b):
        sl = slice(b * cin, (b + 1) * cin)
        acc = jnp.dot(x0t[:, sl], w_ref[0], preferred_element_type=jnp.float32)
        acc = acc + jnp.dot(x1t[:, sl], w_ref[1], preferred_element_type=jnp.float32)
        acc = acc + jnp.dot(x2b[:, sl], w_ref[2], preferred_element_type=jnp.float32)
        outs.append(acc)
    out = jnp.concatenate(outs, axis=1)
    out_ref[...] = out + b_ref[...]


def kernel(x, laplacian, weight, bias):
    n_b, cin, v = x.shape
    k, _, cout = weight.shape
    bc = n_b * cin
    bco = n_b * cout
    tile = _pick_tile(v)
    grid = (v // tile,)

    x0 = jnp.transpose(x, (2, 0, 1)).reshape(v, bc).astype(jnp.bfloat16)
    wb = weight.astype(jnp.bfloat16)
    bt = jnp.tile(bias, n_b)[None, :]

    x1 = pl.pallas_call(
        _stage1,
        grid=grid,
        in_specs=[
            pl.BlockSpec((tile, v), lambda i: (i, 0)),
            pl.BlockSpec((v, bc), lambda i: (0, 0)),
        ],
        out_specs=pl.BlockSpec((tile, bc), lambda i: (i, 0)),
        out_shape=jax.ShapeDtypeStruct((v, bc), jnp.bfloat16),
        compiler_params=pltpu.CompilerParams(
            dimension_semantics=("arbitrary",)),
    )(laplacian, x0)

    out2d = pl.pallas_call(
        functools.partial(_stage2, tile=tile, n_b=n_b, cin=cin),
        grid=grid,
        in_specs=[
            pl.BlockSpec((tile, v), lambda i: (i, 0)),
            pl.BlockSpec((tile, bc), lambda i: (i, 0)),
            pl.BlockSpec((v, bc), lambda i: (0, 0)),
            pl.BlockSpec((k, cin, cout), lambda i: (0, 0, 0)),
            pl.BlockSpec((1, bco), lambda i: (0, 0)),
        ],
        out_specs=pl.BlockSpec((tile, bco), lambda i: (i, 0)),
        out_shape=jax.ShapeDtypeStruct((v, bco), jnp.float32),
        compiler_params=pltpu.CompilerParams(
            dimension_semantics=("arbitrary",)),
    )(laplacian, x0, x1, wb, bt)

    return jnp.transpose(out2d.reshape(v, n_b, cout), (1, 2, 0))
